# baseline (speedup 1.0000x reference)
import functools

import jax
import jax.numpy as jnp
from jax import lax
from jax.experimental import pallas as pl
from jax.experimental.pallas import tpu as pltpu
from jax.experimental.pallas import tpu_sc as plsc

N_NODES = 10000
H = 128
E = 160000
CH = 128
NCHUNKS = E // CH
NSUB = 16
KB = 16
CPT = 80
NCHUNKS_PAD = NSUB * CPT
NGRP = CPT // KB
ROWS_PER_TILE = 624
TAIL_ROWS = N_NODES - NSUB * ROWS_PER_TILE


def _tc_body(bit_ref, hid_ref, w1a_ref, w1b_ref, b1_ref, ef_ref, w2_ref,
             b2_ref, bh_ref, co_ref):
    i = pl.program_id(0)
    acc = jnp.dot(bit_ref[...], w1a_ref[...], preferred_element_type=jnp.float32)
    acc += jnp.dot(hid_ref[...], w1b_ref[...], preferred_element_type=jnp.float32)
    bh_ref[...] = acc + b1_ref[...]
    x = ef_ref[...] * w2_ref[...]
    co = jnp.sum(x, axis=1) + b2_ref[0, 0]
    nrow = co.shape[0] // CH
    co_ref[pl.ds(i * nrow, nrow), :] = co.reshape(nrow, CH)


def _tc_mats(bit2d, hid2d, W1, b1, ef2d, W2, b2):
    grid_n = 25
    blk = 2 * N_NODES // grid_n
    eblk = E // grid_n
    return pl.pallas_call(
        _tc_body,
        grid=(grid_n,),
        in_specs=[
            pl.BlockSpec((blk, H), lambda i: (i, 0)),
            pl.BlockSpec((blk, H), lambda i: (i, 0)),
            pl.BlockSpec((H, H), lambda i: (0, 0)),
            pl.BlockSpec((H, H), lambda i: (0, 0)),
            pl.BlockSpec((1, H), lambda i: (0, 0)),
            pl.BlockSpec((eblk, H), lambda i: (i, 0)),
            pl.BlockSpec((1, H), lambda i: (0, 0)),
            pl.BlockSpec((1, 1), lambda i: (0, 0)),
        ],
        out_specs=(
            pl.BlockSpec((blk, H), lambda i: (i, 0)),
            pl.BlockSpec((NCHUNKS, CH), lambda i: (0, 0)),
        ),
        out_shape=(
            jax.ShapeDtypeStruct((2 * N_NODES, H), jnp.float32),
            jax.ShapeDtypeStruct((NCHUNKS, CH), jnp.float32),
        ),
    )(bit2d, hid2d, W1[:H], W1[H:], b1.reshape(1, H), ef2d,
      W2.reshape(1, H), b2.reshape(1, 1))


def _sc_body(bh_hbm, stage_hbm, coeff_hbm, out_hbm,
             stage_v, coeff_v, tgt_v, gidx_v, rows_a, rows_b, acc_sh,
             sem_g, sem_s):
    c = lax.axis_index("c")
    s = lax.axis_index("s")

    zeros16 = jnp.zeros((16,), jnp.float32)

    def _zero_row(i, carry):
        for j in range(H // 16):
            rows_a[i, pl.ds(16 * j, 16)] = zeros16
        return carry
    lax.fori_loop(0, CH, _zero_row, 0)

    base = s * ROWS_PER_TILE
    nfull = ROWS_PER_TILE // CH
    for q in range(nfull):
        pltpu.sync_copy(rows_a, acc_sh.at[pl.ds(base + q * CH, CH)])
    rem = ROWS_PER_TILE - nfull * CH
    if rem:
        pltpu.sync_copy(rows_a.at[pl.ds(0, rem)],
                        acc_sh.at[pl.ds(base + nfull * CH, rem)])

    @pl.when(s == NSUB - 1)
    def _():
        pltpu.sync_copy(rows_a.at[pl.ds(0, TAIL_ROWS)],
                        acc_sh.at[pl.ds(NSUB * ROWS_PER_TILE, TAIL_ROWS)])
    plsc.subcore_barrier()

    def _scale(buf, q):
        def _scale_group(g, gcarry):
            c16 = coeff_v[q, pl.ds(16 * g, 16)]
            for l in range(16):
                spl = jnp.full((16,), c16[l], jnp.float32)
                e = 16 * g + l
                for j in range(H // 16):
                    sl = pl.ds(16 * j, 16)
                    buf[e, sl] = buf[e, sl] * spl
            return gcarry
        lax.fori_loop(0, CH // 16, _scale_group, 0)

    def _drain(sem):
        pltpu.make_async_copy(bh_hbm.at[gidx_v.at[0]], rows_a, sem).wait()

    def _group(g, carry):
        mb = CPT * s + KB * g
        pltpu.sync_copy(stage_hbm.at[pl.ds(mb, KB)], stage_v)
        pltpu.sync_copy(coeff_hbm.at[pl.ds(mb, KB)], coeff_v)

        def _unpack(i, ucarry):
            for gg in range(CH // 16):
                sl = pl.ds(16 * gg, 16)
                gidx_v[i, sl] = stage_v[i, 0, sl] * 2 + c
                tgt_v[i, sl] = stage_v[i, 1, sl]
            return ucarry
        lax.fori_loop(0, KB, _unpack, 0)

        pltpu.async_copy(bh_hbm.at[gidx_v.at[0]], rows_a, sem_g)

        def _pair(qq, pcarry):
            q0 = 2 * qq
            q1 = 2 * qq + 1
            _drain(sem_g)
            _scale(rows_a, q0)

            @pl.when(qq > 0)
            def _():
                _drain(sem_s)
            pltpu.async_copy(bh_hbm.at[gidx_v.at[q1]], rows_b, sem_g)
            pltpu.async_copy(rows_a, acc_sh.at[tgt_v.at[q0]], sem_s, add=True)

            _drain(sem_g)
            _scale(rows_b, q1)
            _drain(sem_s)

            @pl.when(qq < KB // 2 - 1)
            def _():
                pltpu.async_copy(bh_hbm.at[gidx_v.at[q0 + 2]], rows_a, sem_g)
            pltpu.async_copy(rows_b, acc_sh.at[tgt_v.at[q1]], sem_s, add=True)
            return pcarry

        lax.fori_loop(0, KB // 2, _pair, 0)
        _drain(sem_s)
        return carry

    lax.fori_loop(0, NGRP, _group, 0)

    plsc.subcore_barrier()

    @pl.when(c == 0)
    def _():
        pltpu.sync_copy(acc_sh.at[pl.ds(base, ROWS_PER_TILE)],
                        out_hbm.at[pl.ds(base, ROWS_PER_TILE), 0])

        @pl.when(s == NSUB - 1)
        def _():
            pltpu.sync_copy(acc_sh.at[pl.ds(NSUB * ROWS_PER_TILE, TAIL_ROWS)],
                            out_hbm.at[pl.ds(NSUB * ROWS_PER_TILE, TAIL_ROWS), 0])

    @pl.when(c == 1)
    def _():
        pltpu.sync_copy(acc_sh.at[pl.ds(base, ROWS_PER_TILE)],
                        out_hbm.at[pl.ds(base, ROWS_PER_TILE), 1])

        @pl.when(s == NSUB - 1)
        def _():
            pltpu.sync_copy(acc_sh.at[pl.ds(NSUB * ROWS_PER_TILE, TAIL_ROWS)],
                            out_hbm.at[pl.ds(NSUB * ROWS_PER_TILE, TAIL_ROWS), 1])


@functools.partial(
    pl.kernel,
    mesh=plsc.VectorSubcoreMesh(core_axis_name="c", subcore_axis_name="s"),
    out_type=jax.ShapeDtypeStruct((N_NODES, 2, H), jnp.float32),
    scratch_types=[
        pltpu.VMEM((KB, 2, CH), jnp.int32),
        pltpu.VMEM((KB, CH), jnp.float32),
        pltpu.VMEM((KB, CH), jnp.int32),
        pltpu.VMEM((KB, CH), jnp.int32),
        pltpu.VMEM((CH, H), jnp.float32),
        pltpu.VMEM((CH, H), jnp.float32),
        pltpu.VMEM_SHARED((N_NODES, H), jnp.float32),
        pltpu.SemaphoreType.DMA,
        pltpu.SemaphoreType.DMA,
    ],
)
def _sc_scatter(bh_hbm, stage_hbm, coeff_hbm, out_hbm,
                stage_v, coeff_v, tgt_v, gidx_v, rows_a, rows_b, acc_sh,
                sem_g, sem_s):
    _sc_body(bh_hbm, stage_hbm, coeff_hbm, out_hbm,
             stage_v, coeff_v, tgt_v, gidx_v, rows_a, rows_b, acc_sh,
             sem_g, sem_s)


def kernel(bit_fts, hidden, edge_indices, edge_fts, W1, b1, W2, b2):
    bit2d = bit_fts[0]
    hid2d = hidden[0]
    ef2d = edge_fts[0]
    src = edge_indices[0, :, 0]
    tgt = edge_indices[0, :, 1]

    pad_n = NCHUNKS_PAD * CH - E
    pad_idx = jnp.arange(pad_n, dtype=jnp.int32) % N_NODES
    src2d = jnp.concatenate([src, pad_idx]).reshape(NCHUNKS_PAD, CH)
    tgt2d = jnp.concatenate([tgt, pad_idx]).reshape(NCHUNKS_PAD, CH)

    bh, coeff = _tc_mats(bit2d, hid2d, W1, b1, ef2d, W2, b2)
    coeff2d = jnp.concatenate(
        [coeff.reshape(E), jnp.zeros((pad_n,), jnp.float32)]
    ).reshape(NCHUNKS_PAD, CH)

    stage = jnp.stack([src2d, tgt2d], axis=1)

    out = _sc_scatter(bh, stage, coeff2d)
    return out.reshape(1, 2 * N_NODES, H)


# speedup vs baseline: 1.1246x; 1.1246x over previous
import functools

import jax
import jax.numpy as jnp
from jax import lax
from jax.experimental import pallas as pl
from jax.experimental.pallas import tpu as pltpu
from jax.experimental.pallas import tpu_sc as plsc

N_NODES = 10000
H = 128
E = 160000
CH = 128
NCHUNKS = E // CH
NSUB = 16
KB = 8
CPT = 80
NCHUNKS_PAD = NSUB * CPT
NGRP = CPT // KB
ROWS_PER_TILE = 624
TAIL_ROWS = N_NODES - NSUB * ROWS_PER_TILE


def _tc_body(bit_ref, hid_ref, w1a_ref, w1b_ref, b1_ref, ef_ref, w2_ref,
             b2_ref, bh_ref, co_ref):
    i = pl.program_id(0)
    acc = jnp.dot(bit_ref[...], w1a_ref[...], preferred_element_type=jnp.float32)
    acc += jnp.dot(hid_ref[...], w1b_ref[...], preferred_element_type=jnp.float32)
    bh_ref[...] = acc + b1_ref[...]
    x = ef_ref[...] * w2_ref[...]
    co = jnp.sum(x, axis=1) + b2_ref[0, 0]
    nrow = co.shape[0] // CH
    co_ref[pl.ds(i * nrow, nrow), :] = co.reshape(nrow, CH)


def _tc_mats(bit2d, hid2d, W1, b1, ef2d, W2, b2):
    grid_n = 25
    blk = 2 * N_NODES // grid_n
    eblk = E // grid_n
    return pl.pallas_call(
        _tc_body,
        grid=(grid_n,),
        in_specs=[
            pl.BlockSpec((blk, H), lambda i: (i, 0)),
            pl.BlockSpec((blk, H), lambda i: (i, 0)),
            pl.BlockSpec((H, H), lambda i: (0, 0)),
            pl.BlockSpec((H, H), lambda i: (0, 0)),
            pl.BlockSpec((1, H), lambda i: (0, 0)),
            pl.BlockSpec((eblk, H), lambda i: (i, 0)),
            pl.BlockSpec((1, H), lambda i: (0, 0)),
            pl.BlockSpec((1, 1), lambda i: (0, 0)),
        ],
        out_specs=(
            pl.BlockSpec((blk, H), lambda i: (i, 0)),
            pl.BlockSpec((NCHUNKS, CH), lambda i: (0, 0)),
        ),
        out_shape=(
            jax.ShapeDtypeStruct((2 * N_NODES, H), jnp.float32),
            jax.ShapeDtypeStruct((NCHUNKS, CH), jnp.float32),
        ),
    )(bit2d, hid2d, W1[:H], W1[H:], b1.reshape(1, H), ef2d,
      W2.reshape(1, H), b2.reshape(1, 1))


def _sc_body(bh_hbm, stage_hbm, coeff_hbm, out_hbm,
             stage_v, coeff_v, tgt_v, gidx_v, rows_a, rows_b, acc_sh,
             sem_g, sem_s):
    c = lax.axis_index("c")
    s = lax.axis_index("s")

    zeros16 = jnp.zeros((16,), jnp.float32)

    def _zero_row(i, carry):
        for j in range(H // 16):
            rows_a[i, pl.ds(16 * j, 16)] = zeros16
        return carry
    lax.fori_loop(0, CH, _zero_row, 0)

    base = s * ROWS_PER_TILE
    nfull = ROWS_PER_TILE // CH
    for q in range(nfull):
        pltpu.sync_copy(rows_a, acc_sh.at[pl.ds(base + q * CH, CH)])
    rem = ROWS_PER_TILE - nfull * CH
    if rem:
        pltpu.sync_copy(rows_a.at[pl.ds(0, rem)],
                        acc_sh.at[pl.ds(base + nfull * CH, rem)])

    @pl.when(s == NSUB - 1)
    def _():
        pltpu.sync_copy(rows_a.at[pl.ds(0, TAIL_ROWS)],
                        acc_sh.at[pl.ds(NSUB * ROWS_PER_TILE, TAIL_ROWS)])
    plsc.subcore_barrier()

    def _scale(buf, q):
        def _scale_group(g, gcarry):
            c16 = coeff_v[q, pl.ds(16 * g, 16)]
            for l in range(16):
                spl = jnp.full((16,), c16[l], jnp.float32)
                e = 16 * g + l
                for j in range(H // 16):
                    sl = pl.ds(16 * j, 16)
                    buf[e, sl] = buf[e, sl] * spl
            return gcarry
        lax.fori_loop(0, CH // 16, _scale_group, 0)

    def _drain(sem):
        pltpu.make_async_copy(bh_hbm.at[gidx_v.at[0]], rows_a, sem).wait()

    def _group(g, carry):
        mb = CPT * s + KB * g
        pltpu.sync_copy(stage_hbm.at[pl.ds(mb, KB)], stage_v)
        pltpu.sync_copy(coeff_hbm.at[pl.ds(mb, KB)], coeff_v)

        def _unpack(i, ucarry):
            for gg in range(CH // 16):
                sl = pl.ds(16 * gg, 16)
                gidx_v[i, sl] = stage_v[i, 0, sl] * 2 + c
                tgt_v[i, sl] = stage_v[i, 1, sl]
            return ucarry
        lax.fori_loop(0, KB, _unpack, 0)

        bufs = (rows_a, rows_b)
        pltpu.async_copy(bh_hbm.at[gidx_v.at[0]], bufs[0], sem_g).wait()
        for q in range(KB):
            buf = bufs[q % 2]
            if q + 1 < KB:
                nxt = pltpu.async_copy(bh_hbm.at[gidx_v.at[q + 1]],
                                       bufs[(q + 1) % 2], sem_g)
            _scale(buf, q)
            pltpu.sync_copy(buf, acc_sh.at[tgt_v.at[q]], add=True)
            if q + 1 < KB:
                nxt.wait()
        return carry

    lax.fori_loop(0, NGRP, _group, 0)

    plsc.subcore_barrier()

    @pl.when(c == 0)
    def _():
        pltpu.sync_copy(acc_sh.at[pl.ds(base, ROWS_PER_TILE)],
                        out_hbm.at[pl.ds(base, ROWS_PER_TILE), 0])

        @pl.when(s == NSUB - 1)
        def _():
            pltpu.sync_copy(acc_sh.at[pl.ds(NSUB * ROWS_PER_TILE, TAIL_ROWS)],
                            out_hbm.at[pl.ds(NSUB * ROWS_PER_TILE, TAIL_ROWS), 0])

    @pl.when(c == 1)
    def _():
        pltpu.sync_copy(acc_sh.at[pl.ds(base, ROWS_PER_TILE)],
                        out_hbm.at[pl.ds(base, ROWS_PER_TILE), 1])

        @pl.when(s == NSUB - 1)
        def _():
            pltpu.sync_copy(acc_sh.at[pl.ds(NSUB * ROWS_PER_TILE, TAIL_ROWS)],
                            out_hbm.at[pl.ds(NSUB * ROWS_PER_TILE, TAIL_ROWS), 1])


@functools.partial(
    pl.kernel,
    mesh=plsc.VectorSubcoreMesh(core_axis_name="c", subcore_axis_name="s"),
    out_type=jax.ShapeDtypeStruct((N_NODES, 2, H), jnp.float32),
    scratch_types=[
        pltpu.VMEM((KB, 2, CH), jnp.int32),
        pltpu.VMEM((KB, CH), jnp.float32),
        pltpu.VMEM((KB, CH), jnp.int32),
        pltpu.VMEM((KB, CH), jnp.int32),
        pltpu.VMEM((CH, H), jnp.float32),
        pltpu.VMEM((CH, H), jnp.float32),
        pltpu.VMEM_SHARED((N_NODES, H), jnp.float32),
        pltpu.SemaphoreType.DMA,
        pltpu.SemaphoreType.DMA,
    ],
)
def _sc_scatter(bh_hbm, stage_hbm, coeff_hbm, out_hbm,
                stage_v, coeff_v, tgt_v, gidx_v, rows_a, rows_b, acc_sh,
                sem_g, sem_s):
    _sc_body(bh_hbm, stage_hbm, coeff_hbm, out_hbm,
             stage_v, coeff_v, tgt_v, gidx_v, rows_a, rows_b, acc_sh,
             sem_g, sem_s)


def kernel(bit_fts, hidden, edge_indices, edge_fts, W1, b1, W2, b2):
    bit2d = bit_fts[0]
    hid2d = hidden[0]
    ef2d = edge_fts[0]
    src = edge_indices[0, :, 0]
    tgt = edge_indices[0, :, 1]

    pad_n = NCHUNKS_PAD * CH - E
    pad_idx = jnp.arange(pad_n, dtype=jnp.int32) % N_NODES
    src2d = jnp.concatenate([src, pad_idx]).reshape(NCHUNKS_PAD, CH)
    tgt2d = jnp.concatenate([tgt, pad_idx]).reshape(NCHUNKS_PAD, CH)

    bh, coeff = _tc_mats(bit2d, hid2d, W1, b1, ef2d, W2, b2)
    coeff2d = jnp.concatenate(
        [coeff.reshape(E), jnp.zeros((pad_n,), jnp.float32)]
    ).reshape(NCHUNKS_PAD, CH)

    stage = jnp.stack([src2d, tgt2d], axis=1)

    out = _sc_scatter(bh, stage, coeff2d)
    return out.reshape(1, 2 * N_NODES, H)


# speedup vs baseline: 1.1260x; 1.0013x over previous
import functools

import jax
import jax.numpy as jnp
from jax import lax
from jax.experimental import pallas as pl
from jax.experimental.pallas import tpu as pltpu
from jax.experimental.pallas import tpu_sc as plsc

N_NODES = 10000
H = 128
E = 160000
CH = 128
NCHUNKS = E // CH
NSUB = 16
KB = 8
CPT = 80
NCHUNKS_PAD = NSUB * CPT
NGRP = CPT // KB
ROWS_PER_TILE = 624
TAIL_ROWS = N_NODES - NSUB * ROWS_PER_TILE


def _tc_body(bit_ref, hid_ref, w1a_ref, w1b_ref, b1_ref, ef_ref, w2_ref,
             b2_ref, bh_ref, co_ref):
    i = pl.program_id(0)
    acc = jnp.dot(bit_ref[...], w1a_ref[...], preferred_element_type=jnp.float32)
    acc += jnp.dot(hid_ref[...], w1b_ref[...], preferred_element_type=jnp.float32)
    bh_ref[...] = acc + b1_ref[...]
    x = ef_ref[...] * w2_ref[...]
    co = jnp.sum(x, axis=1) + b2_ref[0, 0]
    nrow = co.shape[0] // CH
    co_ref[pl.ds(i * nrow, nrow), :] = co.reshape(nrow, CH)


def _tc_mats(bit2d, hid2d, W1, b1, ef2d, W2, b2):
    grid_n = 25
    blk = 2 * N_NODES // grid_n
    eblk = E // grid_n
    return pl.pallas_call(
        _tc_body,
        grid=(grid_n,),
        in_specs=[
            pl.BlockSpec((blk, H), lambda i: (i, 0)),
            pl.BlockSpec((blk, H), lambda i: (i, 0)),
            pl.BlockSpec((H, H), lambda i: (0, 0)),
            pl.BlockSpec((H, H), lambda i: (0, 0)),
            pl.BlockSpec((1, H), lambda i: (0, 0)),
            pl.BlockSpec((eblk, H), lambda i: (i, 0)),
            pl.BlockSpec((1, H), lambda i: (0, 0)),
            pl.BlockSpec((1, 1), lambda i: (0, 0)),
        ],
        out_specs=(
            pl.BlockSpec((blk, H), lambda i: (i, 0)),
            pl.BlockSpec((NCHUNKS, CH), lambda i: (0, 0)),
        ),
        out_shape=(
            jax.ShapeDtypeStruct((2 * N_NODES, H), jnp.float32),
            jax.ShapeDtypeStruct((NCHUNKS, CH), jnp.float32),
        ),
    )(bit2d, hid2d, W1[:H], W1[H:], b1.reshape(1, H), ef2d,
      W2.reshape(1, H), b2.reshape(1, 1))


def _sc_body(bh_hbm, stage_hbm, coeff_hbm, out_hbm,
             stage_v, coeff_v, tgt_v, gidx_v, rows_a, rows_b, acc_sh, sem_g):
    c = lax.axis_index("c")
    s = lax.axis_index("s")

    zeros16 = jnp.zeros((16,), jnp.float32)

    def _zero_row(i, carry):
        for j in range(H // 16):
            rows_a[i, pl.ds(16 * j, 16)] = zeros16
        return carry
    lax.fori_loop(0, CH, _zero_row, 0)

    base = s * ROWS_PER_TILE
    nfull = ROWS_PER_TILE // CH
    for q in range(nfull):
        pltpu.sync_copy(rows_a, acc_sh.at[pl.ds(base + q * CH, CH)])
    rem = ROWS_PER_TILE - nfull * CH
    if rem:
        pltpu.sync_copy(rows_a.at[pl.ds(0, rem)],
                        acc_sh.at[pl.ds(base + nfull * CH, rem)])

    @pl.when(s == NSUB - 1)
    def _():
        pltpu.sync_copy(rows_a.at[pl.ds(0, TAIL_ROWS)],
                        acc_sh.at[pl.ds(NSUB * ROWS_PER_TILE, TAIL_ROWS)])
    plsc.subcore_barrier()

    def _scale(buf, q):
        def _scale_group(g, gcarry):
            c16 = coeff_v[q, pl.ds(16 * g, 16)]
            for l in range(16):
                spl = jnp.full((16,), c16[l], jnp.float32)
                e = 16 * g + l
                for j in range(H // 16):
                    sl = pl.ds(16 * j, 16)
                    buf[e, sl] = buf[e, sl] * spl
            return gcarry
        lax.fori_loop(0, CH // 16, _scale_group, 0)

    def _group(g, carry):
        mb = CPT * s + KB * g
        pltpu.sync_copy(stage_hbm.at[pl.ds(mb, KB)], stage_v)
        pltpu.sync_copy(coeff_hbm.at[pl.ds(mb, KB)], coeff_v)

        def _unpack(i, ucarry):
            for gg in range(CH // 16):
                sl = pl.ds(16 * gg, 16)
                gidx_v[i, sl] = stage_v[i, 0, sl] * 2 + c
                tgt_v[i, sl] = stage_v[i, 1, sl]
            return ucarry
        lax.fori_loop(0, KB, _unpack, 0)

        bufs = (rows_a, rows_b)
        pltpu.async_copy(bh_hbm.at[gidx_v.at[0]], bufs[0], sem_g).wait()
        for q in range(KB):
            buf = bufs[q % 2]
            if q + 1 < KB:
                nxt = pltpu.async_copy(bh_hbm.at[gidx_v.at[q + 1]],
                                       bufs[(q + 1) % 2], sem_g)
            _scale(buf, q)
            pltpu.sync_copy(buf, acc_sh.at[tgt_v.at[q]], add=True)
            if q + 1 < KB:
                nxt.wait()
        return carry

    lax.fori_loop(0, NGRP, _group, 0)

    plsc.subcore_barrier()

    @pl.when(c == 0)
    def _():
        pltpu.sync_copy(acc_sh.at[pl.ds(base, ROWS_PER_TILE)],
                        out_hbm.at[pl.ds(base, ROWS_PER_TILE), 0])

        @pl.when(s == NSUB - 1)
        def _():
            pltpu.sync_copy(acc_sh.at[pl.ds(NSUB * ROWS_PER_TILE, TAIL_ROWS)],
                            out_hbm.at[pl.ds(NSUB * ROWS_PER_TILE, TAIL_ROWS), 0])

    @pl.when(c == 1)
    def _():
        pltpu.sync_copy(acc_sh.at[pl.ds(base, ROWS_PER_TILE)],
                        out_hbm.at[pl.ds(base, ROWS_PER_TILE), 1])

        @pl.when(s == NSUB - 1)
        def _():
            pltpu.sync_copy(acc_sh.at[pl.ds(NSUB * ROWS_PER_TILE, TAIL_ROWS)],
                            out_hbm.at[pl.ds(NSUB * ROWS_PER_TILE, TAIL_ROWS), 1])


@functools.partial(
    pl.kernel,
    mesh=plsc.VectorSubcoreMesh(core_axis_name="c", subcore_axis_name="s"),
    out_type=jax.ShapeDtypeStruct((N_NODES, 2, H), jnp.float32),
    scratch_types=[
        pltpu.VMEM((KB, 2, CH), jnp.int32),
        pltpu.VMEM((KB, CH), jnp.float32),
        pltpu.VMEM((KB, CH), jnp.int32),
        pltpu.VMEM((KB, CH), jnp.int32),
        pltpu.VMEM((CH, H), jnp.float32),
        pltpu.VMEM((CH, H), jnp.float32),
        pltpu.VMEM_SHARED((N_NODES, H), jnp.float32),
        pltpu.SemaphoreType.DMA,
    ],
)
def _sc_scatter(bh_hbm, stage_hbm, coeff_hbm, out_hbm,
                stage_v, coeff_v, tgt_v, gidx_v, rows_a, rows_b, acc_sh,
                sem_g):
    _sc_body(bh_hbm, stage_hbm, coeff_hbm, out_hbm,
             stage_v, coeff_v, tgt_v, gidx_v, rows_a, rows_b, acc_sh, sem_g)


def kernel(bit_fts, hidden, edge_indices, edge_fts, W1, b1, W2, b2):
    bit2d = bit_fts[0]
    hid2d = hidden[0]
    ef2d = edge_fts[0]
    src = edge_indices[0, :, 0]
    tgt = edge_indices[0, :, 1]

    pad_n = NCHUNKS_PAD * CH - E
    pad_idx = jnp.arange(pad_n, dtype=jnp.int32) % N_NODES
    src2d = jnp.concatenate([src, pad_idx]).reshape(NCHUNKS_PAD, CH)
    tgt2d = jnp.concatenate([tgt, pad_idx]).reshape(NCHUNKS_PAD, CH)

    bh, coeff = _tc_mats(bit2d, hid2d, W1, b1, ef2d, W2, b2)
    coeff2d = jnp.concatenate(
        [coeff.reshape(E), jnp.zeros((pad_n,), jnp.float32)]
    ).reshape(NCHUNKS_PAD, CH)

    stage = jnp.stack([src2d, tgt2d], axis=1)

    out = _sc_scatter(bh, stage, coeff2d)
    return out.reshape(1, 2 * N_NODES, H)
